# baseline (speedup 1.0000x reference)
import functools

import numpy as np
import jax
import jax.numpy as jnp
from jax import lax
from jax.experimental import pallas as pl
from jax.experimental.pallas import tpu as pltpu
from jax.experimental.pallas import tpu_sc as plsc

ROWS = 128
N = 8192
TOPK = 64
L = 16
NCHUNK = N // L
UNROLL = 16
NCORES = 2
NSUB = 16
NW = NCORES * NSUB
RPW = ROWS // NW
NBINS = 256
SEG = 4
SEGN = N // SEG
SEGCH = NCHUNK // SEG
SEGSZ = SEGN + L
U2 = 4

MINT = np.int32(-2147483648)
M7F = np.int32(0x7FFFFFFF)


def _pc0(m):
    return plsc.all_reduce_population_count(m)[0]


def _skey(x):
    b = lax.bitcast_convert_type(x, jnp.int32)
    return b ^ ((b >> 31) & M7F)


def _bin_scan(hist_v, gs, rank):
    lane = lax.iota(jnp.int32, L)
    sfxg = lax.rev(plsc.cumsum(lax.rev(gs, (0,))), (0,))
    gstar = plsc.all_reduce_population_count(sfxg >= rank) - 1
    carry = jnp.sum(jnp.where(lane > gstar, gs, 0))
    v = hist_v[pl.ds(gstar[0] * L, L)]
    sfx = lax.rev(plsc.cumsum(lax.rev(v, (0,))), (0,)) + carry
    p = plsc.all_reduce_population_count(sfx >= rank) - 1
    bstar = gstar * L + p
    cnt_above = jnp.sum(jnp.where(lane > p, v, 0)) + carry
    return bstar, rank - cnt_above


def _body(in_hbm, out_hbm, xa_v, xb_v, out_v, ci_v, ci2_v, si_v, hist_v,
          gs16_v, sem_a, sem_b, sem_o):
    cid = lax.axis_index("c")
    sid = lax.axis_index("s")
    wid = sid * NCORES + cid
    lane = lax.iota(jnp.int32, L)
    ones = jnp.ones((L,), jnp.int32)
    iz = jnp.zeros((L,), jnp.int32)
    fz = jnp.zeros((L,), jnp.float32)
    row0 = wid * RPW

    def clear_hist():
        for i in range(NBINS // L):
            hist_v[pl.ds(i * L, L)] = iz
        gs16_v[pl.ds(0, L)] = iz

    def zout(i, _):
        out_v[pl.ds(i * L, L)] = fz
        return 0

    def select(x_v):
        def pa(i, accs):
            base = i * (L * UNROLL)
            return tuple(
                jnp.maximum(accs[k], x_v[pl.ds(base + k * L, L)])
                for k in range(UNROLL)
            )

        ninf = jnp.full((L,), -jnp.inf, jnp.float32)
        accs = lax.fori_loop(
            0, NCHUNK // UNROLL, pa, (ninf,) * UNROLL
        )
        g8 = [jnp.maximum(accs[k], accs[k + 8]) for k in range(8)]
        g4 = [jnp.maximum(g8[k], g8[k + 4]) for k in range(4)]
        m01 = jnp.maximum(g4[0], g4[1])
        m23 = jnp.maximum(g4[2], g4[3])
        mx = jnp.max(jnp.maximum(m01, m23))
        tlb = jnp.min(
            jnp.minimum(jnp.minimum(g4[0], g4[1]), jnp.minimum(g4[2], g4[3]))
        )
        tlb_s = jnp.broadcast_to(tlb, (L,))

        def pb(i, offs):
            offs = list(offs)
            for k in range(U2):
                cbase = (i * U2 + k) * L
                for s in range(SEG):
                    base = s * SEGN + cbase
                    x = x_v[pl.ds(base, L)]
                    m = x >= tlb_s
                    plsc.store_compressed(
                        ci_v.at[pl.ds(offs[s], L)], base + lane, mask=m
                    )
                    offs[s] = offs[s] + _pc0(m)
            return tuple(offs)

        offs = lax.fori_loop(
            0, SEGCH // U2, pb,
            tuple(jnp.int32(s * SEGSZ) for s in range(SEG)),
        )
        ns = [offs[s] - s * SEGSZ for s in range(SEG)]

        clear_hist()

        def hseg(s):
            sbase = s * SEGSZ
            n = ns[s]

            def ph(i, _):
                valid = (i * L + lane) < n
                idxv = ci_v[pl.ds(sbase + i * L, L)]
                skey = _skey(plsc.load_gather(x_v, [idxv], mask=valid))
                d0 = ((skey >> 24) & 255) ^ 128
                plsc.addupdate_scatter(hist_v, [d0], ones, mask=valid)
                plsc.addupdate_scatter(gs16_v, [d0 >> 4], ones, mask=valid)
                return 0

            lax.fori_loop(0, (n + L - 1) // L, ph, 0)

        for s in range(SEG):
            hseg(s)
        rank = jnp.full((L,), TOPK, jnp.int32)
        b0, rank = _bin_scan(hist_v, gs16_v[pl.ds(0, L)], rank)

        clear_hist()

        def p30seg(s, carry):
            sbase = s * SEGSZ
            n = ns[s]

            def p30(i, carry):
                goff, coff = carry
                valid = (i * L + lane) < n
                idxv = ci_v[pl.ds(sbase + i * L, L)]
                skey = _skey(plsc.load_gather(x_v, [idxv], mask=valid))
                d0 = ((skey >> 24) & 255) ^ 128
                mg = valid & (d0 > b0)
                me = valid & (d0 == b0)
                d1 = (skey >> 16) & 255
                plsc.addupdate_scatter(hist_v, [d1], ones, mask=me)
                plsc.addupdate_scatter(gs16_v, [d1 >> 4], ones, mask=me)
                plsc.store_compressed(si_v.at[pl.ds(goff, L)], idxv, mask=mg)
                plsc.store_compressed(ci2_v.at[pl.ds(coff, L)], idxv, mask=me)
                return goff + _pc0(mg), coff + _pc0(me)

            return lax.fori_loop(0, (n + L - 1) // L, p30, carry)

        carry = (jnp.int32(0), jnp.int32(0))
        for s in range(SEG):
            carry = p30seg(s, carry)
        ngt, n1 = carry
        b1, rank = _bin_scan(hist_v, gs16_v[pl.ds(0, L)], rank)

        clear_hist()

        def p31(i, carry):
            goff, coff = carry
            valid = (i * L + lane) < n1
            idxv = ci2_v[pl.ds(i * L, L)]
            skey = _skey(plsc.load_gather(x_v, [idxv], mask=valid))
            d1 = (skey >> 16) & 255
            mg = valid & (d1 > b1)
            me = valid & (d1 == b1)
            d2 = (skey >> 8) & 255
            plsc.addupdate_scatter(hist_v, [d2], ones, mask=me)
            plsc.addupdate_scatter(gs16_v, [d2 >> 4], ones, mask=me)
            plsc.store_compressed(si_v.at[pl.ds(goff, L)], idxv, mask=mg)
            plsc.store_compressed(ci2_v.at[pl.ds(coff, L)], idxv, mask=me)
            return goff + _pc0(mg), coff + _pc0(me)

        ngt, n2 = lax.fori_loop(
            0, (n1 + L - 1) // L, p31, (ngt, jnp.int32(0))
        )
        b2, rank = _bin_scan(hist_v, gs16_v[pl.ds(0, L)], rank)

        clear_hist()

        def p32(i, carry):
            goff, coff = carry
            valid = (i * L + lane) < n2
            idxv = ci2_v[pl.ds(i * L, L)]
            skey = _skey(plsc.load_gather(x_v, [idxv], mask=valid))
            d2 = (skey >> 8) & 255
            mg = valid & (d2 > b2)
            me = valid & (d2 == b2)
            d3 = skey & 255
            plsc.addupdate_scatter(hist_v, [d3], ones, mask=me)
            plsc.addupdate_scatter(gs16_v, [d3 >> 4], ones, mask=me)
            plsc.store_compressed(si_v.at[pl.ds(goff, L)], idxv, mask=mg)
            plsc.store_compressed(ci2_v.at[pl.ds(coff, L)], idxv, mask=me)
            return goff + _pc0(mg), coff + _pc0(me)

        ngt, n3 = lax.fori_loop(
            0, (n2 + L - 1) // L, p32, (ngt, jnp.int32(0))
        )
        b3, budget = _bin_scan(hist_v, gs16_v[pl.ds(0, L)], rank)

        def p5(i, carry):
            goff, seen = carry
            valid = (i * L + lane) < n3
            idxv = ci2_v[pl.ds(i * L, L)]
            skey = _skey(plsc.load_gather(x_v, [idxv], mask=valid))
            d3 = skey & 255
            mg = valid & (d3 > b3)
            plsc.store_compressed(si_v.at[pl.ds(goff, L)], idxv, mask=mg)
            goff = goff + _pc0(mg)
            eq = valid & (d3 == b3)
            pos = plsc.cumsum(eq.astype(jnp.int32)) + seen
            sel = eq & (pos <= budget)
            plsc.store_compressed(si_v.at[pl.ds(goff, L)], idxv, mask=sel)
            return goff + _pc0(sel), seen + plsc.all_reduce_population_count(eq)

        lax.fori_loop(0, (n3 + L - 1) // L, p5, (ngt, iz))

        si = [si_v[pl.ds(t * L, L)] for t in range(TOPK // L)]
        es = [
            jnp.exp(plsc.load_gather(x_v, [si[t]]) - mx)
            for t in range(TOPK // L)
        ]
        ssum = jnp.sum((es[0] + es[1]) + (es[2] + es[3]))
        inv = jnp.ones((L,), jnp.float32) / jnp.broadcast_to(ssum, (L,))
        return si, es, inv

    def scatter_gates(si, es, inv):
        for t in range(TOPK // L):
            plsc.store_scatter(out_v, [si[t]], es[t] * inv)

    def rezero(si):
        for t in range(TOPK // L):
            plsc.store_scatter(out_v, [si[t]], fz)

    cp_in0 = pltpu.async_copy(in_hbm.at[row0], xa_v, sem_a)
    lax.fori_loop(0, NCHUNK, zout, 0)
    cp_in0.wait()
    cp_in1 = pltpu.async_copy(in_hbm.at[row0 + 1], xb_v, sem_b)
    si0, es0, inv0 = select(xa_v)
    scatter_gates(si0, es0, inv0)
    cp_out0 = pltpu.async_copy(out_v, out_hbm.at[row0], sem_o)

    cp_in1.wait()
    cp_in2 = pltpu.async_copy(in_hbm.at[row0 + 2], xa_v, sem_a)
    si1, es1, inv1 = select(xb_v)
    cp_out0.wait()
    rezero(si0)
    scatter_gates(si1, es1, inv1)
    cp_out1 = pltpu.async_copy(out_v, out_hbm.at[row0 + 1], sem_o)

    cp_in2.wait()
    cp_in3 = pltpu.async_copy(in_hbm.at[row0 + 3], xb_v, sem_b)
    si2, es2, inv2 = select(xa_v)
    cp_out1.wait()
    rezero(si1)
    scatter_gates(si2, es2, inv2)
    cp_out2 = pltpu.async_copy(out_v, out_hbm.at[row0 + 2], sem_o)

    cp_in3.wait()
    si3, es3, inv3 = select(xb_v)
    cp_out2.wait()
    rezero(si2)
    scatter_gates(si3, es3, inv3)
    pltpu.sync_copy(out_v, out_hbm.at[row0 + 3])


def _make(interpret=False):
    mesh = plsc.VectorSubcoreMesh(
        core_axis_name="c", subcore_axis_name="s",
        num_cores=NCORES, num_subcores=NSUB,
    )
    return pl.kernel(
        _body,
        out_type=jax.ShapeDtypeStruct((ROWS, N), jnp.float32),
        mesh=mesh,
        scratch_types=[
            pltpu.VMEM((N,), jnp.float32),
            pltpu.VMEM((N,), jnp.float32),
            pltpu.VMEM((N,), jnp.float32),
            pltpu.VMEM((SEG * SEGSZ,), jnp.int32),
            pltpu.VMEM((N + 2 * L,), jnp.int32),
            pltpu.VMEM((6 * L,), jnp.int32),
            pltpu.VMEM((NBINS,), jnp.int32),
            pltpu.VMEM((L,), jnp.int32),
            pltpu.SemaphoreType.DMA,
            pltpu.SemaphoreType.DMA,
            pltpu.SemaphoreType.DMA,
        ],
        compiler_params=pltpu.CompilerParams(needs_layout_passes=False),
        interpret=interpret,
    )


_pk = _make()


@jax.jit
def kernel(inputs):
    return _pk(inputs)


# speedup vs baseline: 1.0549x; 1.0549x over previous
import functools

import numpy as np
import jax
import jax.numpy as jnp
from jax import lax
from jax.experimental import pallas as pl
from jax.experimental.pallas import tpu as pltpu
from jax.experimental.pallas import tpu_sc as plsc

ROWS = 128
N = 8192
TOPK = 64
L = 16
NCHUNK = N // L
UNROLL = 16
NCORES = 2
NSUB = 16
NW = NCORES * NSUB
RPW = ROWS // NW
NBINS = 256
SEG = 4
SEGN = N // SEG
SEGCH = NCHUNK // SEG
SEGSZ = SEGN + L
U2 = 4

MINT = np.int32(-2147483648)
M7F = np.int32(0x7FFFFFFF)


def _pc0(m):
    return plsc.all_reduce_population_count(m)[0]


def _skey(x):
    b = lax.bitcast_convert_type(x, jnp.int32)
    return b ^ ((b >> 31) & M7F)


def _bin_scan(hist_v, gs, rank):
    lane = lax.iota(jnp.int32, L)
    sfxg = lax.rev(plsc.cumsum(lax.rev(gs, (0,))), (0,))
    gstar = plsc.all_reduce_population_count(sfxg >= rank) - 1
    carry = jnp.sum(jnp.where(lane > gstar, gs, 0))
    v = hist_v[pl.ds(gstar[0] * L, L)]
    sfx = lax.rev(plsc.cumsum(lax.rev(v, (0,))), (0,)) + carry
    p = plsc.all_reduce_population_count(sfx >= rank) - 1
    bstar = gstar * L + p
    cnt_above = jnp.sum(jnp.where(lane > p, v, 0)) + carry
    return bstar, rank - cnt_above


def _body(in_hbm, out_hbm, x_v, out_v, ci_v, ci2_v, si_v, hist_v, gs16_v):
    cid = lax.axis_index("c")
    sid = lax.axis_index("s")
    wid = sid * NCORES + cid
    lane = lax.iota(jnp.int32, L)
    ones = jnp.ones((L,), jnp.int32)
    iz = jnp.zeros((L,), jnp.int32)
    fz = jnp.zeros((L,), jnp.float32)

    def clear_hist():
        for i in range(NBINS // L):
            hist_v[pl.ds(i * L, L)] = iz
        gs16_v[pl.ds(0, L)] = iz

    def zout(i, _):
        out_v[pl.ds(i * L, L)] = fz
        return 0

    lax.fori_loop(0, NCHUNK, zout, 0)

    def row_body(j, _):
        row = wid * RPW + j
        pltpu.sync_copy(in_hbm.at[row], x_v)

        def pa(i, accs):
            base = i * (L * UNROLL)
            return tuple(
                jnp.maximum(accs[k], x_v[pl.ds(base + k * L, L)])
                for k in range(UNROLL)
            )

        ninf = jnp.full((L,), -jnp.inf, jnp.float32)
        accs = lax.fori_loop(
            0, NCHUNK // UNROLL, pa, (ninf,) * UNROLL
        )
        g8 = [jnp.maximum(accs[k], accs[k + 8]) for k in range(8)]
        g4 = [jnp.maximum(g8[k], g8[k + 4]) for k in range(4)]
        m01 = jnp.maximum(g4[0], g4[1])
        m23 = jnp.maximum(g4[2], g4[3])
        mx = jnp.max(jnp.maximum(m01, m23))
        tlb = jnp.min(
            jnp.minimum(jnp.minimum(g4[0], g4[1]), jnp.minimum(g4[2], g4[3]))
        )
        tlb_s = jnp.broadcast_to(tlb, (L,))

        def pb(i, offs):
            offs = list(offs)
            for k in range(U2):
                cbase = (i * U2 + k) * L
                for s in range(SEG):
                    base = s * SEGN + cbase
                    x = x_v[pl.ds(base, L)]
                    m = x >= tlb_s
                    plsc.store_compressed(
                        ci_v.at[pl.ds(offs[s], L)], base + lane, mask=m
                    )
                    offs[s] = offs[s] + _pc0(m)
            return tuple(offs)

        offs = lax.fori_loop(
            0, SEGCH // U2, pb,
            tuple(jnp.int32(s * SEGSZ) for s in range(SEG)),
        )
        ns = [offs[s] - s * SEGSZ for s in range(SEG)]

        clear_hist()

        def hseg(s):
            sbase = s * SEGSZ
            n = ns[s]

            def ph(i, _):
                valid = (i * L + lane) < n
                idxv = ci_v[pl.ds(sbase + i * L, L)]
                skey = _skey(plsc.load_gather(x_v, [idxv], mask=valid))
                d0 = ((skey >> 24) & 255) ^ 128
                plsc.addupdate_scatter(hist_v, [d0], ones, mask=valid)
                plsc.addupdate_scatter(gs16_v, [d0 >> 4], ones, mask=valid)
                return 0

            lax.fori_loop(0, (n + L - 1) // L, ph, 0)

        for s in range(SEG):
            hseg(s)
        rank = jnp.full((L,), TOPK, jnp.int32)
        b0, rank = _bin_scan(hist_v, gs16_v[pl.ds(0, L)], rank)

        clear_hist()

        def p30seg(s, carry):
            sbase = s * SEGSZ
            n = ns[s]

            def p30(i, carry):
                goff, coff = carry
                valid = (i * L + lane) < n
                idxv = ci_v[pl.ds(sbase + i * L, L)]
                skey = _skey(plsc.load_gather(x_v, [idxv], mask=valid))
                d0 = ((skey >> 24) & 255) ^ 128
                mg = valid & (d0 > b0)
                me = valid & (d0 == b0)
                d1 = (skey >> 16) & 255
                plsc.addupdate_scatter(hist_v, [d1], ones, mask=me)
                plsc.addupdate_scatter(gs16_v, [d1 >> 4], ones, mask=me)
                plsc.store_compressed(si_v.at[pl.ds(goff, L)], idxv, mask=mg)
                plsc.store_compressed(ci2_v.at[pl.ds(coff, L)], idxv, mask=me)
                return goff + _pc0(mg), coff + _pc0(me)

            return lax.fori_loop(0, (n + L - 1) // L, p30, carry)

        carry = (jnp.int32(0), jnp.int32(0))
        for s in range(SEG):
            carry = p30seg(s, carry)
        ngt, n1 = carry
        b1, rank = _bin_scan(hist_v, gs16_v[pl.ds(0, L)], rank)

        clear_hist()

        def p31(i, carry):
            goff, coff = carry
            valid = (i * L + lane) < n1
            idxv = ci2_v[pl.ds(i * L, L)]
            skey = _skey(plsc.load_gather(x_v, [idxv], mask=valid))
            d1 = (skey >> 16) & 255
            mg = valid & (d1 > b1)
            me = valid & (d1 == b1)
            d2 = (skey >> 8) & 255
            plsc.addupdate_scatter(hist_v, [d2], ones, mask=me)
            plsc.addupdate_scatter(gs16_v, [d2 >> 4], ones, mask=me)
            plsc.store_compressed(si_v.at[pl.ds(goff, L)], idxv, mask=mg)
            plsc.store_compressed(ci2_v.at[pl.ds(coff, L)], idxv, mask=me)
            return goff + _pc0(mg), coff + _pc0(me)

        ngt, n2 = lax.fori_loop(
            0, (n1 + L - 1) // L, p31, (ngt, jnp.int32(0))
        )
        b2, rank = _bin_scan(hist_v, gs16_v[pl.ds(0, L)], rank)

        clear_hist()

        def p32(i, carry):
            goff, coff = carry
            valid = (i * L + lane) < n2
            idxv = ci2_v[pl.ds(i * L, L)]
            skey = _skey(plsc.load_gather(x_v, [idxv], mask=valid))
            d2 = (skey >> 8) & 255
            mg = valid & (d2 > b2)
            me = valid & (d2 == b2)
            d3 = skey & 255
            plsc.addupdate_scatter(hist_v, [d3], ones, mask=me)
            plsc.addupdate_scatter(gs16_v, [d3 >> 4], ones, mask=me)
            plsc.store_compressed(si_v.at[pl.ds(goff, L)], idxv, mask=mg)
            plsc.store_compressed(ci2_v.at[pl.ds(coff, L)], idxv, mask=me)
            return goff + _pc0(mg), coff + _pc0(me)

        ngt, n3 = lax.fori_loop(
            0, (n2 + L - 1) // L, p32, (ngt, jnp.int32(0))
        )
        b3, budget = _bin_scan(hist_v, gs16_v[pl.ds(0, L)], rank)

        def p5(i, carry):
            goff, seen = carry
            valid = (i * L + lane) < n3
            idxv = ci2_v[pl.ds(i * L, L)]
            skey = _skey(plsc.load_gather(x_v, [idxv], mask=valid))
            d3 = skey & 255
            mg = valid & (d3 > b3)
            plsc.store_compressed(si_v.at[pl.ds(goff, L)], idxv, mask=mg)
            goff = goff + _pc0(mg)
            eq = valid & (d3 == b3)
            pos = plsc.cumsum(eq.astype(jnp.int32)) + seen
            sel = eq & (pos <= budget)
            plsc.store_compressed(si_v.at[pl.ds(goff, L)], idxv, mask=sel)
            return goff + _pc0(sel), seen + plsc.all_reduce_population_count(eq)

        lax.fori_loop(0, (n3 + L - 1) // L, p5, (ngt, iz))

        si = [si_v[pl.ds(t * L, L)] for t in range(TOPK // L)]
        es = [
            jnp.exp(plsc.load_gather(x_v, [si[t]]) - mx)
            for t in range(TOPK // L)
        ]
        ssum = jnp.sum((es[0] + es[1]) + (es[2] + es[3]))
        inv = jnp.ones((L,), jnp.float32) / jnp.broadcast_to(ssum, (L,))
        for t in range(TOPK // L):
            plsc.store_scatter(out_v, [si[t]], es[t] * inv)
        pltpu.sync_copy(out_v, out_hbm.at[row])
        for t in range(TOPK // L):
            plsc.store_scatter(out_v, [si[t]], fz)
        return 0

    lax.fori_loop(0, RPW, row_body, 0)


def _make(interpret=False):
    mesh = plsc.VectorSubcoreMesh(
        core_axis_name="c", subcore_axis_name="s",
        num_cores=NCORES, num_subcores=NSUB,
    )
    return pl.kernel(
        _body,
        out_type=jax.ShapeDtypeStruct((ROWS, N), jnp.float32),
        mesh=mesh,
        scratch_types=[
            pltpu.VMEM((N,), jnp.float32),
            pltpu.VMEM((N,), jnp.float32),
            pltpu.VMEM((SEG * SEGSZ,), jnp.int32),
            pltpu.VMEM((N + 2 * L,), jnp.int32),
            pltpu.VMEM((6 * L,), jnp.int32),
            pltpu.VMEM((NBINS,), jnp.int32),
            pltpu.VMEM((L,), jnp.int32),
        ],
        compiler_params=pltpu.CompilerParams(needs_layout_passes=False),
        interpret=interpret,
    )


_pk = _make()


@jax.jit
def kernel(inputs):
    return _pk(inputs)
